# baseline (speedup 1.0000x reference)
import functools

import jax
import jax.numpy as jnp
from jax import lax
from jax.experimental import pallas as pl
from jax.experimental.pallas import tpu as pltpu
from jax.experimental.pallas import tpu_sc as plsc

L = 16
CHUNK = 128
PACK = 4


@functools.cache
def _build(B, D, NC, NS):
    NW = NC * NS
    b_per_w = B // NW
    n_chunks = b_per_w // CHUNK
    groups_per_chunk = CHUNK // L
    mesh = plsc.VectorSubcoreMesh(core_axis_name="c", subcore_axis_name="s")

    @functools.partial(
        pl.kernel,
        mesh=mesh,
        compiler_params=pltpu.CompilerParams(
            needs_layout_passes=False, use_tc_tiling_on_sc=False),
        out_type=(
            jax.ShapeDtypeStruct((NW, b_per_w), jnp.float32),
            jax.ShapeDtypeStruct((NW, b_per_w), jnp.float32),
        ),
        scratch_types=[
            pltpu.VMEM((n_chunks, CHUNK), jnp.int32),
            pltpu.VMEM((n_chunks, CHUNK), jnp.int32),
            pltpu.VMEM((n_chunks, CHUNK), jnp.int32),
            pltpu.VMEM((n_chunks, CHUNK), jnp.int32),
            pltpu.VMEM((n_chunks, CHUNK), jnp.int32),
            pltpu.VMEM((n_chunks, CHUNK), jnp.int32),
            pltpu.VMEM((CHUNK, PACK * D), jnp.float32),
            pltpu.VMEM((CHUNK, PACK * D), jnp.float32),
            pltpu.VMEM((CHUNK, PACK * D), jnp.float32),
            pltpu.VMEM((CHUNK, PACK * D), jnp.float32),
            pltpu.VMEM((CHUNK, PACK * D), jnp.float32),
            pltpu.VMEM((CHUNK, PACK * D), jnp.float32),
            pltpu.VMEM((b_per_w,), jnp.float32),
            pltpu.VMEM((b_per_w,), jnp.float32),
            pltpu.SemaphoreType.DMA,
            pltpu.SemaphoreType.DMA,
        ],
    )
    def bpr_kernel(uid_hbm, pid_hbm, nid_hbm, utab, atab, pos_out, neg_out,
                   uid, pid, nid, uq, pq, nq, u0, u1, p0, p1, n0, n1,
                   posv, negv, sem0, sem1):
        wid = lax.axis_index("s") * NC + lax.axis_index("c")
        pltpu.sync_copy(uid_hbm.at[wid], uid)
        pltpu.sync_copy(pid_hbm.at[wid], pid)
        pltpu.sync_copy(nid_hbm.at[wid], nid)
        for j in range(n_chunks):
            for s in range(CHUNK // L):
                sl = pl.ds(s * L, L)
                uq[j, sl] = jnp.right_shift(uid[j, sl], 2)
                pq[j, sl] = jnp.right_shift(pid[j, sl], 2)
                nq[j, sl] = jnp.right_shift(nid[j, sl], 2)

        bufs = ((u0, p0, n0, sem0), (u1, p1, n1, sem1))

        def fire(j, buf):
            ub, pb, nb, sem = bufs[buf]
            return (
                pltpu.async_copy(utab.at[uq.at[j]], ub, sem),
                pltpu.async_copy(atab.at[pq.at[j]], pb, sem),
                pltpu.async_copy(atab.at[nq.at[j]], nb, sem),
            )

        lane = lax.iota(jnp.int32, L)
        inflight = fire(0, 0)
        for j in range(n_chunks):
            buf = j % 2
            ub, pb, nb, _ = bufs[buf]
            for c in inflight:
                c.wait()
            if j + 1 < n_chunks:
                inflight = fire(j + 1, 1 - buf)

            def group(g, _, j=j, ub=ub, pb=pb, nb=nb):
                sl = pl.ds(g * L, L)
                ubase = jnp.left_shift(jnp.bitwise_and(uid[j, sl], 3), 5)
                pbase = jnp.left_shift(jnp.bitwise_and(pid[j, sl], 3), 5)
                nbase = jnp.left_shift(jnp.bitwise_and(nid[j, sl], 3), 5)
                rows = lane + g * L
                pacc = jnp.zeros((L,), jnp.float32)
                nacc = jnp.zeros((L,), jnp.float32)
                for d in range(D):
                    u = plsc.load_gather(ub, [rows, ubase + d])
                    p = plsc.load_gather(pb, [rows, pbase + d])
                    nn = plsc.load_gather(nb, [rows, nbase + d])
                    pacc = pacc + u * p
                    nacc = nacc + u * nn
                out_sl = pl.ds(j * CHUNK + g * L, L)
                posv[out_sl] = pacc
                negv[out_sl] = nacc
                return _

            lax.fori_loop(0, groups_per_chunk, group, None)
        pltpu.sync_copy(posv, pos_out.at[wid])
        pltpu.sync_copy(negv, neg_out.at[wid])

    return bpr_kernel


def kernel(user_ids, pos_action_ids, neg_action_ids, user_table, action_table):
    B = user_ids.shape[0]
    D = user_table.shape[1]
    info = plsc.get_sparse_core_info()
    NC, NS = info.num_cores, info.num_subcores
    NW = NC * NS
    b_per_w = B // NW
    n_chunks = b_per_w // CHUNK
    uid = user_ids.astype(jnp.int32).reshape(NW, n_chunks, CHUNK)
    pid = pos_action_ids.astype(jnp.int32).reshape(NW, n_chunks, CHUNK)
    nid = neg_action_ids.astype(jnp.int32).reshape(NW, n_chunks, CHUNK)
    utab = user_table.reshape(-1, PACK * D)
    atab = action_table.reshape(-1, PACK * D)
    pos, neg = _build(B, D, NC, NS)(uid, pid, nid, utab, atab)
    return pos.reshape(B), neg.reshape(B)


# speedup vs baseline: 2.6907x; 2.6907x over previous
import functools

import jax
import jax.numpy as jnp
from jax import lax
from jax.experimental import pallas as pl
from jax.experimental.pallas import tpu as pltpu
from jax.experimental.pallas import tpu_sc as plsc

L = 16
CHUNK = 128
UCHUNK = 16
PACK = 4
LANES = 128


@functools.cache
def _build(B, D, NC, NS):
    NW = NC * NS
    b_per_w = B // NW
    n_chunks = b_per_w // CHUNK
    n_uchunks = b_per_w // UCHUNK
    mesh = plsc.VectorSubcoreMesh(core_axis_name="c", subcore_axis_name="s")

    @functools.partial(
        pl.kernel,
        mesh=mesh,
        compiler_params=pltpu.CompilerParams(
            needs_layout_passes=False, use_tc_tiling_on_sc=True),
        out_type=(
            jax.ShapeDtypeStruct((NW, b_per_w), jnp.float32),
            jax.ShapeDtypeStruct((NW, b_per_w), jnp.float32),
        ),
        scratch_types=[
            pltpu.VMEM((b_per_w,), jnp.int32),
            pltpu.VMEM((b_per_w,), jnp.int32),
            pltpu.VMEM((b_per_w,), jnp.int32),
            pltpu.VMEM((n_chunks, CHUNK), jnp.int32),
            pltpu.VMEM((n_chunks, CHUNK), jnp.int32),
            pltpu.VMEM((UCHUNK, D, LANES), jnp.float32),
            pltpu.VMEM((CHUNK, PACK * D), jnp.float32),
            pltpu.VMEM((CHUNK, PACK * D), jnp.float32),
            pltpu.VMEM((b_per_w,), jnp.float32),
            pltpu.VMEM((b_per_w,), jnp.float32),
            pltpu.SemaphoreType.DMA,
            pltpu.SemaphoreType.DMA,
        ],
    )
    def bpr_kernel(uid_hbm, pid_hbm, nid_hbm, utab_t, atab_p,
                   pos_out, neg_out,
                   uidv, pidv, nidv, pq, nq, uwin, prow, nrow,
                   posv, negv, semu, sema):
        wid = lax.axis_index("s") * NC + lax.axis_index("c")
        pltpu.sync_copy(uid_hbm.at[wid], uidv)
        pltpu.sync_copy(pid_hbm.at[wid], pidv)
        pltpu.sync_copy(nid_hbm.at[wid], nidv)
        for j in range(n_chunks):
            for s in range(CHUNK // L):
                sl = pl.ds(j * CHUNK + s * L, L)
                dsl = pl.ds(s * L, L)
                pq[j, dsl] = jnp.right_shift(pidv[sl], 2)
                nq[j, dsl] = jnp.right_shift(nidv[sl], 2)

        lane = lax.iota(jnp.int32, L)
        per_chunk = CHUNK // UCHUNK
        for c in range(n_chunks):
            hp = pltpu.async_copy(atab_p.at[pq.at[c]], prow, sema)
            hn = pltpu.async_copy(atab_p.at[nq.at[c]], nrow, sema)

            def uchunk(uc, _, c=c):
                base = c * CHUNK + uc * UCHUNK
                gsl = pl.ds(base, L)
                vblk = jnp.right_shift(uidv[gsl], 7)
                handles = []
                for k in range(UCHUNK):
                    s = jnp.max(jnp.where(lane == k, vblk, 0))
                    blk = pl.multiple_of(s * LANES, 128)
                    handles.append(pltpu.async_copy(
                        utab_t.at[:, pl.ds(blk, LANES)], uwin.at[k], semu))
                for h in handles:
                    h.wait()
                ulane = jnp.bitwise_and(uidv[gsl], 127)
                pbase = jnp.left_shift(jnp.bitwise_and(pidv[gsl], 3), 5)
                nbase = jnp.left_shift(jnp.bitwise_and(nidv[gsl], 3), 5)
                arow = lane + uc * UCHUNK
                pacc = jnp.zeros((L,), jnp.float32)
                nacc = jnp.zeros((L,), jnp.float32)
                for d in range(D):
                    dsplat = jnp.full((L,), d, jnp.int32)
                    u = plsc.load_gather(uwin, [lane, dsplat, ulane])
                    p = plsc.load_gather(prow, [arow, pbase + d])
                    nn = plsc.load_gather(nrow, [arow, nbase + d])
                    pacc = pacc + u * p
                    nacc = nacc + u * nn
                posv[gsl] = pacc
                negv[gsl] = nacc
                return _

            hp.wait()
            hn.wait()
            lax.fori_loop(0, per_chunk, uchunk, None)
        pltpu.sync_copy(posv, pos_out.at[wid])
        pltpu.sync_copy(negv, neg_out.at[wid])

    return bpr_kernel


def kernel(user_ids, pos_action_ids, neg_action_ids, user_table, action_table):
    B = user_ids.shape[0]
    D = user_table.shape[1]
    info = plsc.get_sparse_core_info()
    NC, NS = info.num_cores, info.num_subcores
    NW = NC * NS
    b_per_w = B // NW
    uid = user_ids.astype(jnp.int32).reshape(NW, b_per_w)
    pid = pos_action_ids.astype(jnp.int32).reshape(NW, b_per_w)
    nid = neg_action_ids.astype(jnp.int32).reshape(NW, b_per_w)
    utab_t = user_table.T
    atab_p = action_table.reshape(-1, PACK * D)
    pos, neg = _build(B, D, NC, NS)(uid, pid, nid, utab_t, atab_p)
    return pos.reshape(B), neg.reshape(B)
